# baseline (device time: 23084 ns/iter reference)
import jax
import jax.numpy as jnp
from jax import lax
from jax.experimental import pallas as pl
from jax.experimental.pallas import tpu as pltpu

N_DEV = 4


def kernel(x, dy, gamma):
    m_per, d = x.shape

    def body(x_ref, dy_ref, gamma_ref, out_ref, send_buf, comm_ref,
             send_sems, recv_sems):
        my_pos = lax.axis_index("i")

        xf = x_ref[...].astype(jnp.float32)
        dyf = dy_ref[...].astype(jnp.float32)
        mu = jnp.mean(xf, axis=1, keepdims=True)
        xc = xf - mu
        var = jnp.mean(xc * xc, axis=1, keepdims=True)
        xhat = xc * lax.rsqrt(var + 1e-5)
        send_buf[0, :] = jnp.sum(dyf * xhat, axis=0)
        send_buf[1, :] = jnp.sum(dyf, axis=0)

        barrier_sem = pltpu.get_barrier_semaphore()
        for k in range(1, N_DEV):
            pl.semaphore_signal(
                barrier_sem, inc=1,
                device_id=((my_pos + k) % N_DEV,),
                device_id_type=pl.DeviceIdType.MESH,
            )
        pl.semaphore_wait(barrier_sem, N_DEV - 1)

        rdmas = []
        for k in range(1, N_DEV):
            slot = N_DEV - 1 - k
            rdma = pltpu.make_async_remote_copy(
                src_ref=send_buf,
                dst_ref=comm_ref.at[slot],
                send_sem=send_sems.at[slot],
                recv_sem=recv_sems.at[slot],
                device_id=((my_pos + k) % N_DEV,),
                device_id_type=pl.DeviceIdType.MESH,
            )
            rdma.start()
            rdmas.append(rdma)
        for rdma in rdmas:
            rdma.wait()

        out_ref[...] = (send_buf[...] + comm_ref[0] + comm_ref[1]
                        + comm_ref[2])

    return pl.pallas_call(
        body,
        out_shape=jax.ShapeDtypeStruct((2, d), jnp.float32),
        in_specs=[
            pl.BlockSpec(memory_space=pltpu.VMEM),
            pl.BlockSpec(memory_space=pltpu.VMEM),
            pl.BlockSpec(memory_space=pltpu.VMEM),
        ],
        out_specs=pl.BlockSpec(memory_space=pltpu.VMEM),
        scratch_shapes=[
            pltpu.VMEM((2, d), jnp.float32),
            pltpu.VMEM((N_DEV - 1, 2, d), jnp.float32),
            pltpu.SemaphoreType.DMA((N_DEV - 1,)),
            pltpu.SemaphoreType.DMA((N_DEV - 1,)),
        ],
        compiler_params=pltpu.CompilerParams(collective_id=0),
    )(x, dy, gamma)


# device time: 22806 ns/iter; 1.0122x vs baseline; 1.0122x over previous
import jax
import jax.numpy as jnp
from jax import lax
from jax.experimental import pallas as pl
from jax.experimental.pallas import tpu as pltpu

N_DEV = 4
M_BLK = 256


def kernel(x, dy, gamma):
    m_per, d = x.shape
    grid = m_per // M_BLK

    def body(x_ref, dy_ref, gamma_ref, out_ref, acc, comm_ref,
             send_sems, recv_sems):
        step = pl.program_id(0)

        xf = x_ref[...]
        dyf = dy_ref[...]
        mu = jnp.mean(xf, axis=1, keepdims=True)
        xc = xf - mu
        var = jnp.mean(xc * xc, axis=1, keepdims=True)
        xhat = xc * lax.rsqrt(var + 1e-5)
        pg = jnp.sum(dyf * xhat, axis=0)
        pb = jnp.sum(dyf, axis=0)

        @pl.when(step == 0)
        def _():
            acc[0, :] = pg
            acc[1, :] = pb

        @pl.when(step > 0)
        def _():
            acc[0, :] = acc[0, :] + pg
            acc[1, :] = acc[1, :] + pb

        @pl.when(step == grid - 1)
        def _():
            my_pos = lax.axis_index("i")

            barrier_sem = pltpu.get_barrier_semaphore()
            for k in range(1, N_DEV):
                pl.semaphore_signal(
                    barrier_sem, inc=1,
                    device_id=((my_pos + k) % N_DEV,),
                    device_id_type=pl.DeviceIdType.MESH,
                )
            pl.semaphore_wait(barrier_sem, N_DEV - 1)

            rdmas = []
            for k in range(1, N_DEV):
                slot = N_DEV - 1 - k
                rdma = pltpu.make_async_remote_copy(
                    src_ref=acc,
                    dst_ref=comm_ref.at[slot],
                    send_sem=send_sems.at[slot],
                    recv_sem=recv_sems.at[slot],
                    device_id=((my_pos + k) % N_DEV,),
                    device_id_type=pl.DeviceIdType.MESH,
                )
                rdma.start()
                rdmas.append(rdma)
            for rdma in rdmas:
                rdma.wait()

            out_ref[...] = (acc[...] + comm_ref[0] + comm_ref[1]
                            + comm_ref[2])

    return pl.pallas_call(
        body,
        grid=(grid,),
        out_shape=jax.ShapeDtypeStruct((2, d), jnp.float32),
        in_specs=[
            pl.BlockSpec((M_BLK, d), lambda i: (i, 0)),
            pl.BlockSpec((M_BLK, d), lambda i: (i, 0)),
            pl.BlockSpec(memory_space=pl.ANY),
        ],
        out_specs=pl.BlockSpec((2, d), lambda i: (0, 0)),
        scratch_shapes=[
            pltpu.VMEM((2, d), jnp.float32),
            pltpu.VMEM((N_DEV - 1, 2, d), jnp.float32),
            pltpu.SemaphoreType.DMA((N_DEV - 1,)),
            pltpu.SemaphoreType.DMA((N_DEV - 1,)),
        ],
        compiler_params=pltpu.CompilerParams(collective_id=0),
    )(x, dy, gamma)


# device time: 20683 ns/iter; 1.1161x vs baseline; 1.1026x over previous
import jax
import jax.numpy as jnp
from jax import lax
from jax.experimental import pallas as pl
from jax.experimental.pallas import tpu as pltpu

N_DEV = 4
M_BLK = 256


def kernel(x, dy, gamma):
    m_per, d = x.shape
    grid = m_per // M_BLK

    def body(x_ref, dy_ref, gamma_ref, out_ref, acc, comm_ref,
             send_sems, recv_sems):
        step = pl.program_id(0)

        xf = x_ref[...]
        dyf = dy_ref[...]
        pg = jnp.sum(xf, axis=0)
        pb = jnp.sum(dyf, axis=0)

        @pl.when(step == 0)
        def _():
            acc[0, :] = pg
            acc[1, :] = pb

        @pl.when(step > 0)
        def _():
            acc[0, :] = acc[0, :] + pg
            acc[1, :] = acc[1, :] + pb

        @pl.when(step == grid - 1)
        def _():
            my_pos = lax.axis_index("i")

            barrier_sem = pltpu.get_barrier_semaphore()
            for k in range(1, N_DEV):
                pl.semaphore_signal(
                    barrier_sem, inc=1,
                    device_id=((my_pos + k) % N_DEV,),
                    device_id_type=pl.DeviceIdType.MESH,
                )
            pl.semaphore_wait(barrier_sem, N_DEV - 1)

            rdmas = []
            for k in range(1, N_DEV):
                slot = N_DEV - 1 - k
                rdma = pltpu.make_async_remote_copy(
                    src_ref=acc,
                    dst_ref=comm_ref.at[slot],
                    send_sem=send_sems.at[slot],
                    recv_sem=recv_sems.at[slot],
                    device_id=((my_pos + k) % N_DEV,),
                    device_id_type=pl.DeviceIdType.MESH,
                )
                rdma.start()
                rdmas.append(rdma)
            for rdma in rdmas:
                rdma.wait()

            out_ref[...] = (acc[...] + comm_ref[0] + comm_ref[1]
                            + comm_ref[2])

    return pl.pallas_call(
        body,
        grid=(grid,),
        out_shape=jax.ShapeDtypeStruct((2, d), jnp.float32),
        in_specs=[
            pl.BlockSpec((M_BLK, d), lambda i: (i, 0)),
            pl.BlockSpec((M_BLK, d), lambda i: (i, 0)),
            pl.BlockSpec(memory_space=pl.ANY),
        ],
        out_specs=pl.BlockSpec((2, d), lambda i: (0, 0)),
        scratch_shapes=[
            pltpu.VMEM((2, d), jnp.float32),
            pltpu.VMEM((N_DEV - 1, 2, d), jnp.float32),
            pltpu.SemaphoreType.DMA((N_DEV - 1,)),
            pltpu.SemaphoreType.DMA((N_DEV - 1,)),
        ],
        compiler_params=pltpu.CompilerParams(collective_id=0),
    )(x, dy, gamma)


# device time: 18788 ns/iter; 1.2287x vs baseline; 1.1009x over previous
import os

import jax
import jax.numpy as jnp
from jax import lax
from jax.experimental import pallas as pl
from jax.experimental.pallas import tpu as pltpu

N_DEV = 4
CH = int(os.environ.get("CH", "256"))
DIAG = os.environ.get("DIAG", "")


def kernel(x, dy, gamma):
    m_per, d = x.shape
    nch = m_per // CH

    def body(x_hbm, dy_hbm, gamma_hbm, out_ref, xbuf, dybuf, acc, comm_ref,
             copy_sems, send_sems, recv_sems):
        my_pos = lax.axis_index("i")

        if DIAG != "2":
            barrier_sem = pltpu.get_barrier_semaphore()
            for k in range(1, N_DEV):
                pl.semaphore_signal(
                    barrier_sem, inc=1,
                    device_id=((my_pos + k) % N_DEV,),
                    device_id_type=pl.DeviceIdType.MESH,
                )

        def chunk_copies(c):
            rows = pl.ds(c * CH, CH)
            return [
                pltpu.make_async_copy(x_hbm.at[rows, :], xbuf.at[rows, :],
                                      copy_sems.at[0, c]),
                pltpu.make_async_copy(dy_hbm.at[rows, :], dybuf.at[rows, :],
                                      copy_sems.at[1, c]),
            ]

        if DIAG != "3":
            for c in range(nch):
                for cp in chunk_copies(c):
                    cp.start()

            for c in range(nch):
                for cp in chunk_copies(c):
                    cp.wait()
                rows = pl.ds(c * CH, CH)
                xf = xbuf[rows, :]
                dyf = dybuf[rows, :]
                if DIAG:
                    pg = jnp.sum(xf, axis=0)
                    pb = jnp.sum(dyf, axis=0)
                else:
                    mu = jnp.mean(xf, axis=1, keepdims=True)
                    xc = xf - mu
                    var = jnp.mean(xc * xc, axis=1, keepdims=True)
                    xhat = xc * lax.rsqrt(var + 1e-5)
                    pg = jnp.sum(dyf * xhat, axis=0)
                    pb = jnp.sum(dyf, axis=0)
                if c == 0:
                    acc[0, :] = pg
                    acc[1, :] = pb
                else:
                    acc[0, :] = acc[0, :] + pg
                    acc[1, :] = acc[1, :] + pb
        else:
            acc[...] = jnp.zeros((2, d), jnp.float32)

        if DIAG == "2":
            out_ref[...] = acc[...]
            return

        pl.semaphore_wait(barrier_sem, N_DEV - 1)

        rdmas = []
        for k in range(1, N_DEV):
            slot = N_DEV - 1 - k
            rdma = pltpu.make_async_remote_copy(
                src_ref=acc,
                dst_ref=comm_ref.at[slot],
                send_sem=send_sems.at[slot],
                recv_sem=recv_sems.at[slot],
                device_id=((my_pos + k) % N_DEV,),
                device_id_type=pl.DeviceIdType.MESH,
            )
            rdma.start()
            rdmas.append(rdma)
        for rdma in rdmas:
            rdma.wait()

        out_ref[...] = acc[...] + comm_ref[0] + comm_ref[1] + comm_ref[2]

    return pl.pallas_call(
        body,
        out_shape=jax.ShapeDtypeStruct((2, d), jnp.float32),
        in_specs=[
            pl.BlockSpec(memory_space=pl.ANY),
            pl.BlockSpec(memory_space=pl.ANY),
            pl.BlockSpec(memory_space=pl.ANY),
        ],
        out_specs=pl.BlockSpec(memory_space=pltpu.VMEM),
        scratch_shapes=[
            pltpu.VMEM((m_per, d), jnp.float32),
            pltpu.VMEM((m_per, d), jnp.float32),
            pltpu.VMEM((2, d), jnp.float32),
            pltpu.VMEM((N_DEV - 1, 2, d), jnp.float32),
            pltpu.SemaphoreType.DMA((2, nch)),
            pltpu.SemaphoreType.DMA((N_DEV - 1,)),
            pltpu.SemaphoreType.DMA((N_DEV - 1,)),
        ],
        compiler_params=(
            pltpu.CompilerParams(vmem_limit_bytes=60 * 1024 * 1024)
            if DIAG == "2"
            else pltpu.CompilerParams(collective_id=0,
                                      vmem_limit_bytes=60 * 1024 * 1024)),
    )(x, dy, gamma)
